# baseline (speedup 1.0000x reference)
import functools

import jax
import jax.numpy as jnp
from jax.experimental import pallas as pl


def _band_body(x_ref, w_ref, b_ref, o_ref, *, deltas, blk, act):
    acc = None
    for k, d in enumerate(deltas):
        t = jax.lax.dot_general(
            x_ref[d:d + blk, :], w_ref[k],
            (((1,), (0,)), ((), ())), preferred_element_type=jnp.float32)
        acc = t if acc is None else acc + t
    acc = acc + b_ref[...]
    if act == 'relu':
        acc = jnp.maximum(acc, 0.0)
    elif act == 'sigmoid':
        acc = jax.nn.sigmoid(acc)
    o_ref[...] = acc.astype(o_ref.dtype)


def _band_conv(xf, w4, bias, deltas, act, out_dtype, blk=1024):
    m, c = xf.shape
    n = w4.shape[2]
    grid = -(-m // blk)
    halo = -(-max(deltas) // 8) * 8
    mpad = grid * blk + halo
    xf = jnp.pad(xf, ((0, mpad - m), (0, 0)))
    bias2 = bias.reshape(1, n).astype(jnp.float32)
    out = pl.pallas_call(
        functools.partial(_band_body, deltas=deltas, blk=blk, act=act),
        grid=(grid,),
        in_specs=[
            pl.BlockSpec((pl.Element(blk + halo), pl.Element(c)),
                         lambda i: (i * blk, 0)),
            pl.BlockSpec((4, c, n), lambda i: (0, 0, 0)),
            pl.BlockSpec((1, n), lambda i: (0, 0)),
        ],
        out_specs=pl.BlockSpec((blk, n), lambda i: (i, 0)),
        out_shape=jax.ShapeDtypeStruct((grid * blk, n), out_dtype),
    )(xf, w4, bias2)
    return out[:m]


def _fwd_w4(w):
    o, c = w.shape[0], w.shape[1]
    wr = w.reshape(o, c, 2, 2, 2, 2)
    return wr.transpose(2, 4, 3, 5, 1, 0).reshape(4, 4 * c, o)


def _tr_w4(w):
    c, o = w.shape[0], w.shape[1]
    wr = w[:, :, ::-1, ::-1].reshape(c, o, 2, 2, 2, 2)
    return wr.transpose(2, 4, 0, 3, 5, 1).reshape(4, c, 4 * o)


def _vq_body(z_ref, cb_ref, zq_ref):
    z = z_ref[...]
    cb = cb_ref[...]
    z_norm = jnp.sum(z * z, axis=1, keepdims=True)
    cb_norm = jnp.sum(cb * cb, axis=1)[None, :]
    d = (z_norm + cb_norm) - 2.0 * jax.lax.dot_general(
        z, cb, (((1,), (1,)), ((), ())), preferred_element_type=jnp.float32)
    d_min = jnp.min(d, axis=1, keepdims=True)
    k = cb.shape[0]
    iota = jax.lax.broadcasted_iota(jnp.int32, d.shape, 1)
    masked_iota = jnp.where(d == d_min, iota, k)
    idx = jnp.min(masked_iota, axis=1, keepdims=True)
    onehot = (iota == idx).astype(jnp.float32)
    zq_ref[...] = jax.lax.dot_general(
        onehot, cb, (((1,), (0,)), ((), ())),
        preferred_element_type=jnp.float32)


def _vq_lookup(z_flat, codebook, blk=1024):
    n, d = z_flat.shape
    k = codebook.shape[0]
    return pl.pallas_call(
        _vq_body,
        grid=(n // blk,),
        in_specs=[
            pl.BlockSpec((blk, d), lambda i: (i, 0)),
            pl.BlockSpec((k, d), lambda i: (0, 0)),
        ],
        out_specs=pl.BlockSpec((blk, d), lambda i: (i, 0)),
        out_shape=jax.ShapeDtypeStruct((n, d), jnp.float32),
    )(z_flat, codebook)


def kernel(x, enc_w1, enc_b1, enc_w2, enc_b2, codebook,
           dec_w1, dec_b1, dec_w2, dec_b2):
    b = x.shape[0]

    xp = jnp.pad(x, ((0, 0), (0, 0), (1, 1), (1, 1)))
    xf1 = (xp.reshape(b, 3, 193, 2, 193, 2)
           .transpose(0, 2, 4, 3, 5, 1).reshape(b * 193 * 193, 12))
    p1 = _band_conv(xf1, _fwd_w4(enc_w1), enc_b1,
                    (0, 1, 193, 194), 'relu', jnp.float32)
    h = p1.reshape(b, 193, 193, 64)[:, :192, :192, :]

    hp = jnp.pad(h, ((0, 0), (1, 1), (1, 1), (0, 0)))
    xf2 = (hp.reshape(b, 97, 2, 97, 2, 64)
           .transpose(0, 1, 3, 2, 4, 5).reshape(b * 97 * 97, 256))
    p2 = _band_conv(xf2, _fwd_w4(enc_w2), enc_b2,
                    (0, 1, 97, 98), 'relu', jnp.float32)
    z_e = p2.reshape(b, 97, 97, 64)[:, :96, :96, :]

    z_e_nchw = z_e.transpose(0, 3, 1, 2)
    z_flat = z_e_nchw.reshape(-1, 64)
    z_q_flat = _vq_lookup(z_flat, codebook)
    z_q = z_q_flat.reshape(b, 64, 96, 96)

    zq_nhwc = z_q.transpose(0, 2, 3, 1)
    zqp = jnp.pad(zq_nhwc, ((0, 0), (1, 1), (1, 1), (0, 0)))
    xfd1 = zqp.reshape(b * 98 * 98, 64).astype(jnp.bfloat16)
    pd1 = _band_conv(xfd1, _tr_w4(dec_w1).astype(jnp.bfloat16),
                     jnp.tile(dec_b1, 4), (0, 1, 98, 99), 'relu',
                     jnp.bfloat16)
    pd1 = pd1.reshape(b, 98, 98, 4, 64)
    quad = jnp.stack([pd1[:, 0:96, 0:96, 0], pd1[:, 0:96, 1:97, 1],
                      pd1[:, 1:97, 0:96, 2], pd1[:, 1:97, 1:97, 3]],
                     axis=-2)
    h2 = (quad.reshape(b, 96, 96, 2, 2, 64).transpose(0, 1, 3, 2, 4, 5)
          .reshape(b, 192, 192, 64))

    h2p = jnp.pad(h2, ((0, 0), (1, 1), (1, 1), (0, 0)))
    xfd2 = h2p.reshape(b * 194 * 194, 64)
    pd2 = _band_conv(xfd2, _tr_w4(dec_w2).astype(jnp.bfloat16),
                     jnp.tile(dec_b2, 4), (0, 1, 194, 195), 'sigmoid',
                     jnp.float32)
    pd2 = pd2.reshape(b, 194, 194, 4, 3)
    quad2 = jnp.stack([pd2[:, 0:192, 0:192, 0], pd2[:, 0:192, 1:193, 1],
                       pd2[:, 1:193, 0:192, 2], pd2[:, 1:193, 1:193, 3]],
                      axis=-2)
    x_recon = (quad2.reshape(b, 192, 192, 2, 2, 3).transpose(0, 1, 3, 2, 4, 5)
               .reshape(b, 384, 384, 3).transpose(0, 3, 1, 2))

    return (x_recon, z_q)


# speedup vs baseline: 2.6397x; 2.6397x over previous
import functools

import jax
import jax.numpy as jnp
from jax.experimental import pallas as pl
from jax.experimental.pallas import tpu as pltpu


def _conv2d(x, w, b, stride, pad):
    out = jax.lax.conv_general_dilated(
        x, w, (stride, stride), ((pad, pad), (pad, pad)),
        dimension_numbers=('NCHW', 'OIHW', 'NCHW'))
    return out + b[None, :, None, None]


def _conv_transpose2d(x, w, b, stride, pad):
    k = w.shape[2]
    w_conv = jnp.transpose(jnp.flip(w, (2, 3)), (1, 0, 2, 3))
    p = k - 1 - pad
    out = jax.lax.conv_general_dilated(
        x, w_conv, (1, 1), ((p, p), (p, p)), lhs_dilation=(stride, stride),
        dimension_numbers=('NCHW', 'OIHW', 'NCHW'))
    return out + b[None, :, None, None]


def _vq_body(z_ref, cb_ref, zq_ref):
    z = z_ref[...]
    cb = cb_ref[...]
    z_norm = jnp.sum(z * z, axis=1, keepdims=True)
    cb_norm = jnp.sum(cb * cb, axis=1)[None, :]
    d = (z_norm + cb_norm) - 2.0 * jax.lax.dot_general(
        z, cb, (((1,), (1,)), ((), ())), preferred_element_type=jnp.float32)
    d_min = jnp.min(d, axis=1, keepdims=True)
    k = cb.shape[0]
    iota = jax.lax.broadcasted_iota(jnp.int32, d.shape, 1)
    masked_iota = jnp.where(d == d_min, iota, k)
    idx = jnp.min(masked_iota, axis=1, keepdims=True)
    onehot = (iota == idx).astype(jnp.float32)
    zq_ref[...] = jax.lax.dot_general(
        onehot, cb, (((1,), (0,)), ((), ())),
        preferred_element_type=jnp.float32)


@functools.partial(jax.jit, static_argnames=('blk',))
def _vq_lookup(z_flat, codebook, blk=1024):
    n, d = z_flat.shape
    k = codebook.shape[0]
    grid = n // blk
    return pl.pallas_call(
        _vq_body,
        grid=(grid,),
        in_specs=[
            pl.BlockSpec((blk, d), lambda i: (i, 0)),
            pl.BlockSpec((k, d), lambda i: (0, 0)),
        ],
        out_specs=pl.BlockSpec((blk, d), lambda i: (i, 0)),
        out_shape=jax.ShapeDtypeStruct((n, d), jnp.float32),
    )(z_flat, codebook)


def kernel(x, enc_w1, enc_b1, enc_w2, enc_b2, codebook,
           dec_w1, dec_b1, dec_w2, dec_b2):
    h = jax.nn.relu(_conv2d(x, enc_w1, enc_b1, 2, 1))
    z_e = jax.nn.relu(_conv2d(h, enc_w2, enc_b2, 2, 1))
    z_e_flat = jnp.reshape(z_e, (-1, z_e.shape[1]))
    z_q = _vq_lookup(z_e_flat, codebook).reshape(z_e.shape)
    zq16 = z_q.astype(jnp.bfloat16)
    h2 = jax.nn.relu(_conv_transpose2d(zq16, dec_w1.astype(jnp.bfloat16),
                                       dec_b1, 2, 1))
    x_recon = jax.nn.sigmoid(_conv_transpose2d(
        h2.astype(jnp.bfloat16), dec_w2.astype(jnp.bfloat16), dec_b2, 2, 1))
    x_recon = x_recon.astype(jnp.float32)
    return (x_recon, z_q)


# speedup vs baseline: 3.7389x; 1.4164x over previous
import functools

import jax
import jax.numpy as jnp
from jax.experimental import pallas as pl
from jax.experimental.pallas import tpu as pltpu


def _conv2d(x, w, b, stride, pad):
    out = jax.lax.conv_general_dilated(
        x, w, (stride, stride), ((pad, pad), (pad, pad)),
        dimension_numbers=('NCHW', 'OIHW', 'NCHW'))
    return out + b[None, :, None, None]


def _conv_transpose2d(x, w, b, stride, pad):
    k = w.shape[2]
    w_conv = jnp.transpose(jnp.flip(w, (2, 3)), (1, 0, 2, 3))
    p = k - 1 - pad
    out = jax.lax.conv_general_dilated(
        x, w_conv, (1, 1), ((p, p), (p, p)), lhs_dilation=(stride, stride),
        dimension_numbers=('NCHW', 'OIHW', 'NCHW'))
    return out + b[None, :, None, None]


def _vq_body(z_ref, cb_ref, zq_ref):
    z = z_ref[...]
    cb = cb_ref[...]
    z_norm = jnp.sum(z * z, axis=1, keepdims=True)
    cb_norm = jnp.sum(cb * cb, axis=1)[None, :]
    d = (z_norm + cb_norm) - 2.0 * jax.lax.dot_general(
        z, cb, (((1,), (1,)), ((), ())), preferred_element_type=jnp.float32)
    d_min = jnp.min(d, axis=1, keepdims=True)
    k = cb.shape[0]
    iota = jax.lax.broadcasted_iota(jnp.int32, d.shape, 1)
    masked_iota = jnp.where(d == d_min, iota, k)
    idx = jnp.min(masked_iota, axis=1, keepdims=True)
    onehot = (iota == idx).astype(jnp.float32)
    zq_ref[...] = jax.lax.dot_general(
        onehot, cb, (((1,), (0,)), ((), ())),
        preferred_element_type=jnp.float32)


@functools.partial(jax.jit, static_argnames=('blk',))
def _vq_lookup(z_flat, codebook, blk=1024):
    n, d = z_flat.shape
    k = codebook.shape[0]
    grid = n // blk
    return pl.pallas_call(
        _vq_body,
        grid=(grid,),
        in_specs=[
            pl.BlockSpec((blk, d), lambda i: (i, 0)),
            pl.BlockSpec((k, d), lambda i: (0, 0)),
        ],
        out_specs=pl.BlockSpec((blk, d), lambda i: (i, 0)),
        out_shape=jax.ShapeDtypeStruct((n, d), jnp.float32),
    )(z_flat, codebook)


def _tr_phase_conv(x_nhwc, w, bias, act):
    bsz, hh, ww, c = x_nhwc.shape
    o = w.shape[1]
    xp = jnp.pad(x_nhwc, ((0, 0), (1, 1), (1, 1), (0, 0))).astype(jnp.bfloat16)
    wr = w[:, :, ::-1, ::-1].reshape(c, o, 2, 2, 2, 2)
    wp = wr.transpose(2, 4, 0, 3, 5, 1).reshape(2, 2, c, 4 * o)
    p = jax.lax.conv_general_dilated(
        xp, wp.astype(jnp.bfloat16), (1, 1), 'VALID',
        dimension_numbers=('NHWC', 'HWIO', 'NHWC'),
        preferred_element_type=jnp.float32)
    p = p + jnp.tile(bias, 4)
    if act == 'relu':
        p = jnp.maximum(p, 0.0)
    else:
        p = jax.nn.sigmoid(p)
    p = p.reshape(bsz, hh + 1, ww + 1, 4, o)
    quad = jnp.stack([p[:, 0:hh, 0:ww, 0], p[:, 0:hh, 1:ww + 1, 1],
                      p[:, 1:hh + 1, 0:ww, 2], p[:, 1:hh + 1, 1:ww + 1, 3]],
                     axis=-2)
    return (quad.reshape(bsz, hh, ww, 2, 2, o).transpose(0, 1, 3, 2, 4, 5)
            .reshape(bsz, 2 * hh, 2 * ww, o))


def kernel(x, enc_w1, enc_b1, enc_w2, enc_b2, codebook,
           dec_w1, dec_b1, dec_w2, dec_b2):
    h = jax.nn.relu(_conv2d(x, enc_w1, enc_b1, 2, 1))
    z_e = jax.nn.relu(_conv2d(h, enc_w2, enc_b2, 2, 1))
    z_e_flat = jnp.reshape(z_e, (-1, z_e.shape[1]))
    z_q = _vq_lookup(z_e_flat, codebook).reshape(z_e.shape)
    b = x.shape[0]
    h2 = _tr_phase_conv(z_q.transpose(0, 2, 3, 1), dec_w1, dec_b1, 'relu')
    x_rec_nhwc = _tr_phase_conv(h2, dec_w2, dec_b2, 'sigmoid')
    x_recon = x_rec_nhwc.transpose(0, 3, 1, 2)
    return (x_recon, z_q)


# speedup vs baseline: 3.7406x; 1.0004x over previous
import functools

import jax
import jax.numpy as jnp
from jax.experimental import pallas as pl
from jax.experimental.pallas import tpu as pltpu


def _conv2d(x, w, b, stride, pad):
    out = jax.lax.conv_general_dilated(
        x, w, (stride, stride), ((pad, pad), (pad, pad)),
        dimension_numbers=('NCHW', 'OIHW', 'NCHW'))
    return out + b[None, :, None, None]


def _conv_transpose2d(x, w, b, stride, pad):
    k = w.shape[2]
    w_conv = jnp.transpose(jnp.flip(w, (2, 3)), (1, 0, 2, 3))
    p = k - 1 - pad
    out = jax.lax.conv_general_dilated(
        x, w_conv, (1, 1), ((p, p), (p, p)), lhs_dilation=(stride, stride),
        dimension_numbers=('NCHW', 'OIHW', 'NCHW'))
    return out + b[None, :, None, None]


def _vq_body(z_ref, cb_ref, zq_ref):
    z = z_ref[...]
    cb = cb_ref[...]
    z_norm = jnp.sum(z * z, axis=1, keepdims=True)
    cb_norm = jnp.sum(cb * cb, axis=1)[None, :]
    d = (z_norm + cb_norm) - 2.0 * jax.lax.dot_general(
        z, cb, (((1,), (1,)), ((), ())), preferred_element_type=jnp.float32)
    d_min = jnp.min(d, axis=1, keepdims=True)
    k = cb.shape[0]
    iota = jax.lax.broadcasted_iota(jnp.int32, d.shape, 1)
    masked_iota = jnp.where(d == d_min, iota, k)
    idx = jnp.min(masked_iota, axis=1, keepdims=True)
    onehot = (iota == idx).astype(jnp.float32)
    zq_ref[...] = jax.lax.dot_general(
        onehot, cb, (((1,), (0,)), ((), ())),
        preferred_element_type=jnp.float32)


@functools.partial(jax.jit, static_argnames=('blk',))
def _vq_lookup(z_flat, codebook, blk=1024):
    n, d = z_flat.shape
    k = codebook.shape[0]
    grid = n // blk
    return pl.pallas_call(
        _vq_body,
        grid=(grid,),
        in_specs=[
            pl.BlockSpec((blk, d), lambda i: (i, 0)),
            pl.BlockSpec((k, d), lambda i: (0, 0)),
        ],
        out_specs=pl.BlockSpec((blk, d), lambda i: (i, 0)),
        out_shape=jax.ShapeDtypeStruct((n, d), jnp.float32),
    )(z_flat, codebook)


def _tr_phase_conv(x_nhwc, w, bias, act):
    bsz, hh, ww, c = x_nhwc.shape
    o = w.shape[1]
    xp = jnp.pad(x_nhwc, ((0, 0), (1, 1), (1, 1), (0, 0))).astype(jnp.bfloat16)
    wr = w[:, :, ::-1, ::-1].reshape(c, o, 2, 2, 2, 2)
    wp = wr.transpose(2, 4, 0, 3, 5, 1).reshape(2, 2, c, 4 * o)
    p = jax.lax.conv_general_dilated(
        xp, wp.astype(jnp.bfloat16), (1, 1), 'VALID',
        dimension_numbers=('NHWC', 'HWIO', 'NHWC'),
        preferred_element_type=jnp.float32)
    p = p + jnp.tile(bias, 4)
    if act == 'relu':
        p = jnp.maximum(p, 0.0)
    else:
        p = jax.nn.sigmoid(p)
    p = p.reshape(bsz, hh + 1, ww + 1, 4, o)
    quad = jnp.stack([p[:, 0:hh, 0:ww, 0], p[:, 0:hh, 1:ww + 1, 1],
                      p[:, 1:hh + 1, 0:ww, 2], p[:, 1:hh + 1, 1:ww + 1, 3]],
                     axis=-2)
    return (quad.reshape(bsz, hh, ww, 2, 2, o).transpose(0, 1, 3, 2, 4, 5)
            .reshape(bsz, 2 * hh, 2 * ww, o))


def kernel(x, enc_w1, enc_b1, enc_w2, enc_b2, codebook,
           dec_w1, dec_b1, dec_w2, dec_b2):
    x_nhwc = x.transpose(0, 2, 3, 1)
    h = jax.nn.relu(jax.lax.conv_general_dilated(
        x_nhwc, enc_w1.transpose(2, 3, 1, 0), (2, 2), ((1, 1), (1, 1)),
        dimension_numbers=('NHWC', 'HWIO', 'NHWC')) + enc_b1)
    z_e_nhwc = jax.nn.relu(jax.lax.conv_general_dilated(
        h, enc_w2.transpose(2, 3, 1, 0), (2, 2), ((1, 1), (1, 1)),
        dimension_numbers=('NHWC', 'HWIO', 'NHWC')) + enc_b2)
    z_e = z_e_nhwc.transpose(0, 3, 1, 2)
    z_e_flat = jnp.reshape(z_e, (-1, z_e.shape[1]))
    z_q = _vq_lookup(z_e_flat, codebook).reshape(z_e.shape)
    b = x.shape[0]
    h2 = _tr_phase_conv(z_q.transpose(0, 2, 3, 1), dec_w1, dec_b1, 'relu')
    x_rec_nhwc = _tr_phase_conv(h2, dec_w2, dec_b2, 'sigmoid')
    x_recon = x_rec_nhwc.transpose(0, 3, 1, 2)
    return (x_recon, z_q)
